# Initial kernel scaffold; baseline (speedup 1.0000x reference)
import jax
import jax.numpy as jnp
from jax.experimental import pallas as pl


def kernel(input_ids, token_type_ids, word_emb, pos_emb, type_emb, ln_gamma, ln_beta):
    raise NotImplementedError("write your pallas kernel here")


# speedup vs baseline: 5.1789x; 5.1789x over previous
import functools

import jax
import jax.numpy as jnp
from jax import lax
from jax.experimental import pallas as pl
from jax.experimental.pallas import tpu as pltpu
from jax.experimental.pallas import tpu_sc as plsc

HIDDEN = 128
EPS = 1e-12

NC = 2
NS = 16
NW = NC * NS


def _sc_gather(ids_flat, table, ch):
    n = ids_flat.shape[0]
    per_w = n // NW
    steps = per_w // ch
    mesh = plsc.VectorSubcoreMesh(core_axis_name="c", subcore_axis_name="s")

    @functools.partial(
        pl.kernel,
        out_type=jax.ShapeDtypeStruct((n, HIDDEN), jnp.float32),
        mesh=mesh,
        scratch_types=[
            pltpu.VMEM((ch,), jnp.int32),
            pltpu.VMEM((ch, HIDDEN), jnp.float32),
            pltpu.SemaphoreType.DMA,
        ],
    )
    def gather_k(ids_hbm, table_hbm, out_hbm, idx_v, rows_v, sem):
        wid = lax.axis_index("s") * NC + lax.axis_index("c")

        def body(g, carry):
            base = wid * per_w + g * ch
            pltpu.sync_copy(ids_hbm.at[pl.ds(base, ch)], idx_v)
            pltpu.async_copy(table_hbm.at[idx_v], rows_v, sem).wait()
            pltpu.sync_copy(rows_v, out_hbm.at[pl.ds(base, ch)])
            return carry

        lax.fori_loop(0, steps, body, 0)

    return gather_k(ids_flat, table)


def _tc_ln_body(wg_ref, tt_ref, pos_ref, type_ref, gam_ref, bet_ref, out_ref):
    tt = tt_ref[...].astype(jnp.float32)
    t0 = type_ref[0:1, :]
    dt = type_ref[1:2, :] - t0
    x = wg_ref[...] + pos_ref[...] + (t0 + tt * dt)
    mu = jnp.mean(x, axis=-1, keepdims=True)
    xc = x - mu
    var = jnp.mean(xc * xc, axis=-1, keepdims=True)
    y = xc * lax.rsqrt(var + EPS)
    out_ref[...] = y * gam_ref[...] + bet_ref[...]


def kernel(input_ids, token_type_ids, word_emb, pos_emb, type_emb, ln_gamma, ln_beta):
    b, s = input_ids.shape
    n = b * s
    ids_flat = input_ids.reshape(n).astype(jnp.int32)
    tt2 = token_type_ids.reshape(n, 1).astype(jnp.int32)

    wg = _sc_gather(ids_flat, word_emb, ch=512)

    t_blk = s
    out = pl.pallas_call(
        _tc_ln_body,
        grid=(n // t_blk,),
        in_specs=[
            pl.BlockSpec((t_blk, HIDDEN), lambda i: (i, 0)),
            pl.BlockSpec((t_blk, 1), lambda i: (i, 0)),
            pl.BlockSpec((t_blk, HIDDEN), lambda i: (0, 0)),
            pl.BlockSpec((2, HIDDEN), lambda i: (0, 0)),
            pl.BlockSpec((1, HIDDEN), lambda i: (0, 0)),
            pl.BlockSpec((1, HIDDEN), lambda i: (0, 0)),
        ],
        out_specs=pl.BlockSpec((t_blk, HIDDEN), lambda i: (i, 0)),
        out_shape=jax.ShapeDtypeStruct((n, HIDDEN), jnp.float32),
    )(wg, tt2, pos_emb, type_emb, ln_gamma.reshape(1, HIDDEN), ln_beta.reshape(1, HIDDEN))
    return out.reshape(b, s, HIDDEN)


# speedup vs baseline: 11.4318x; 2.2074x over previous
import functools

import jax
import jax.numpy as jnp
from jax import lax
from jax.experimental import pallas as pl
from jax.experimental.pallas import tpu as pltpu
from jax.experimental.pallas import tpu_sc as plsc

HIDDEN = 128
EPS = 1e-12

NC = 2
NS = 16
NW = NC * NS


def _sc_gather(ids_flat, table, ch):
    n = ids_flat.shape[0]
    per_w = n // NW
    steps = per_w // ch
    mesh = plsc.VectorSubcoreMesh(core_axis_name="c", subcore_axis_name="s")

    @functools.partial(
        pl.kernel,
        out_type=jax.ShapeDtypeStruct((n, HIDDEN), jnp.float32),
        mesh=mesh,
        scratch_types=[
            pltpu.VMEM((ch,), jnp.int32),
            pltpu.VMEM((ch, HIDDEN), jnp.float32),
            pltpu.SemaphoreType.DMA,
        ],
    )
    def gather_k(ids_hbm, table_hbm, out_hbm, idx_v, rows_v, sem):
        wid = lax.axis_index("s") * NC + lax.axis_index("c")

        def body(g, carry):
            base = wid * per_w + g * ch
            pltpu.sync_copy(ids_hbm.at[pl.ds(base, ch)], idx_v)
            pltpu.async_copy(table_hbm.at[idx_v], rows_v, sem).wait()
            pltpu.sync_copy(rows_v, out_hbm.at[pl.ds(base, ch)])
            return carry

        lax.fori_loop(0, steps, body, 0)

    return gather_k(ids_flat, table)


def _tc_ln_body(wg_ref, tt_ref, pos_ref, type_ref, gam_ref, bet_ref, out_ref):
    tt = tt_ref[...].astype(jnp.float32)[:, :, None]
    t0 = type_ref[0:1, :]
    dt = (type_ref[1:2, :] - t0)[None, :, :]
    x = wg_ref[...] + pos_ref[...][None, :, :] + (t0[None, :, :] + tt * dt)
    mu = jnp.mean(x, axis=-1, keepdims=True)
    xc = x - mu
    var = jnp.mean(xc * xc, axis=-1, keepdims=True)
    y = xc * lax.rsqrt(var + EPS)
    out_ref[...] = y * gam_ref[...][None, :, :] + bet_ref[...][None, :, :]


def kernel(input_ids, token_type_ids, word_emb, pos_emb, type_emb, ln_gamma, ln_beta):
    b, s = input_ids.shape
    n = b * s
    ids_flat = input_ids.reshape(n).astype(jnp.int32)
    tt2 = token_type_ids.astype(jnp.int32)

    wg = _sc_gather(ids_flat, word_emb, ch=512)

    nb = 8
    out = pl.pallas_call(
        _tc_ln_body,
        grid=(b // nb,),
        in_specs=[
            pl.BlockSpec((nb, s, HIDDEN), lambda i: (i, 0, 0)),
            pl.BlockSpec((nb, s), lambda i: (i, 0)),
            pl.BlockSpec((s, HIDDEN), lambda i: (0, 0)),
            pl.BlockSpec((2, HIDDEN), lambda i: (0, 0)),
            pl.BlockSpec((1, HIDDEN), lambda i: (0, 0)),
            pl.BlockSpec((1, HIDDEN), lambda i: (0, 0)),
        ],
        out_specs=pl.BlockSpec((nb, s, HIDDEN), lambda i: (i, 0, 0)),
        out_shape=jax.ShapeDtypeStruct((b, s, HIDDEN), jnp.float32),
    )(wg.reshape(b, s, HIDDEN), tt2, pos_emb, type_emb,
      ln_gamma.reshape(1, HIDDEN), ln_beta.reshape(1, HIDDEN))
    return out


# speedup vs baseline: 12.5468x; 1.0975x over previous
import functools

import jax
import jax.numpy as jnp
from jax import lax
from jax.experimental import pallas as pl
from jax.experimental.pallas import tpu as pltpu
from jax.experimental.pallas import tpu_sc as plsc

HIDDEN = 128
EPS = 1e-12

NC = 2
NS = 16
NW = NC * NS


def _sc_gather(ids_flat, table, ch):
    n = ids_flat.shape[0]
    per_w = n // NW
    steps = per_w // ch
    mesh = plsc.VectorSubcoreMesh(core_axis_name="c", subcore_axis_name="s")

    @functools.partial(
        pl.kernel,
        out_type=jax.ShapeDtypeStruct((n, HIDDEN), jnp.float32),
        mesh=mesh,
        scratch_types=[
            pltpu.VMEM((ch,), jnp.int32),
            pltpu.VMEM((ch, HIDDEN), jnp.float32),
            pltpu.SemaphoreType.DMA,
        ],
    )
    def gather_k(ids_hbm, table_hbm, out_hbm, idx_v, rows_v, sem):
        wid = lax.axis_index("s") * NC + lax.axis_index("c")

        def body(g, carry):
            base = wid * per_w + g * ch
            pltpu.sync_copy(ids_hbm.at[pl.ds(base, ch)], idx_v)
            pltpu.async_copy(table_hbm.at[idx_v], rows_v, sem).wait()
            pltpu.sync_copy(rows_v, out_hbm.at[pl.ds(base, ch)])
            return carry

        lax.fori_loop(0, steps, body, 0)

    return gather_k(ids_flat, table)


def _tc_ln_body(wg_ref, tt_ref, pos_ref, type_ref, gam_ref, bet_ref, out_ref):
    tt = tt_ref[...].astype(jnp.float32)[:, :, None]
    t0 = type_ref[0:1, :]
    dt = (type_ref[1:2, :] - t0)[None, :, :]
    x = wg_ref[...] + pos_ref[...][None, :, :] + (t0[None, :, :] + tt * dt)
    mu = jnp.mean(x, axis=-1, keepdims=True)
    xc = x - mu
    var = jnp.mean(xc * xc, axis=-1, keepdims=True)
    y = xc * lax.rsqrt(var + EPS)
    out_ref[...] = y * gam_ref[...][None, :, :] + bet_ref[...][None, :, :]


def kernel(input_ids, token_type_ids, word_emb, pos_emb, type_emb, ln_gamma, ln_beta):
    b, s = input_ids.shape
    n = b * s
    ids_flat = input_ids.reshape(n).astype(jnp.int32)
    tt2 = token_type_ids.astype(jnp.int32)

    wg = _sc_gather(ids_flat, word_emb, ch=512)

    nb = 32
    out = pl.pallas_call(
        _tc_ln_body,
        grid=(b // nb,),
        in_specs=[
            pl.BlockSpec((nb, s, HIDDEN), lambda i: (i, 0, 0)),
            pl.BlockSpec((nb, s), lambda i: (i, 0)),
            pl.BlockSpec((s, HIDDEN), lambda i: (0, 0)),
            pl.BlockSpec((2, HIDDEN), lambda i: (0, 0)),
            pl.BlockSpec((1, HIDDEN), lambda i: (0, 0)),
            pl.BlockSpec((1, HIDDEN), lambda i: (0, 0)),
        ],
        out_specs=pl.BlockSpec((nb, s, HIDDEN), lambda i: (i, 0, 0)),
        out_shape=jax.ShapeDtypeStruct((b, s, HIDDEN), jnp.float32),
    )(wg.reshape(b, s, HIDDEN), tt2, pos_emb, type_emb,
      ln_gamma.reshape(1, HIDDEN), ln_beta.reshape(1, HIDDEN))
    return out
